# baseline (speedup 1.0000x reference)
import functools

import jax
import jax.numpy as jnp
from jax import lax
from jax.experimental import pallas as pl
from jax.experimental.pallas import tpu as pltpu
from jax.experimental.pallas import tpu_sc as plsc

N_NODES = 10000
N_EDGES = 320000
HIDDEN = 128

NC = 2
NS = 16
NW = NC * NS
LANES = 16

CHUNK = 80
EPT = N_EDGES // NW
NCH = EPT // CHUNK
EPT_DEG = N_EDGES // NS
NCH_DEG = EPT_DEG // CHUNK
E_PAD = 327680
EPT_E = E_PAD // NW
CHUNK_E = 64
NCH_E = EPT_E // CHUNK_E
N_PAD = 10240
ROWS_PT = N_PAD // NS
DEG_W = 16

_MESH = plsc.VectorSubcoreMesh(core_axis_name="c", subcore_axis_name="s")
_f32 = jnp.float32


@functools.partial(
    pl.kernel,
    out_type=jax.ShapeDtypeStruct((NC, N_PAD, DEG_W), _f32),
    mesh=_MESH,
    scratch_types=[
        pltpu.VMEM((NCH_DEG, CHUNK), jnp.int32),
        pltpu.VMEM((CHUNK, DEG_W), _f32),
        pltpu.VMEM_SHARED((N_PAD, DEG_W), _f32),
    ],
    compiler_params=pltpu.CompilerParams(use_tc_tiling_on_sc=False),
)
def _sc_degrees(ei, z16, ones, out, idx_v, ones_v, acc):
    c = lax.axis_index("c")
    s = lax.axis_index("s")
    pltpu.sync_copy(ei.at[c, s], idx_v)
    pltpu.sync_copy(ones, ones_v)
    pltpu.sync_copy(z16, acc.at[pl.ds(s * ROWS_PT, ROWS_PT)])
    plsc.subcore_barrier()

    @pl.loop(0, NCH_DEG)
    def _(j):
        pltpu.sync_copy(ones_v, acc.at[idx_v.at[j]], add=True)

    plsc.subcore_barrier()
    pltpu.sync_copy(acc.at[pl.ds(s * ROWS_PT, ROWS_PT)],
                    out.at[c, pl.ds(s * ROWS_PT, ROWS_PT)])


NBUF = 5
HALF = HIDDEN // NC


@functools.partial(
    pl.kernel,
    out_type=jax.ShapeDtypeStruct((NC, N_PAD, HALF), _f32),
    mesh=_MESH,
    scratch_types=[
        pltpu.VMEM((NCH_DEG, CHUNK), jnp.int32),
        pltpu.VMEM((NCH_DEG, CHUNK), jnp.int32),
        pltpu.VMEM((NBUF, CHUNK, HALF), _f32),
        pltpu.VMEM_SHARED((N_PAD, HALF), _f32),
        pltpu.SemaphoreType.DMA((NBUF,)),
        pltpu.SemaphoreType.DMA((NBUF,)),
    ],
    compiler_params=pltpu.CompilerParams(use_tc_tiling_on_sc=False),
)
def _sc_aggregate(g, ei, z, out, idx_s, idx_d, rows, acc, semg, sems):
    c = lax.axis_index("c")
    s = lax.axis_index("s")
    gh = g.at[c]
    pltpu.sync_copy(ei.at[0, s], idx_s)
    pltpu.sync_copy(ei.at[1, s], idx_d)
    pltpu.sync_copy(z, acc.at[pl.ds(s * ROWS_PT, ROWS_PT)])
    plsc.subcore_barrier()

    for b in range(NBUF):
        pltpu.async_copy(gh.at[idx_s.at[b]], rows.at[b], semg.at[b])

    @pl.loop(0, NCH_DEG - NBUF, step=NBUF)
    def _(j):
        for b in range(NBUF):
            ch = j + b
            pltpu.make_async_copy(gh.at[idx_s.at[ch]], rows.at[b],
                                  semg.at[b]).wait()
            pltpu.async_copy(rows.at[b], acc.at[idx_d.at[ch]], sems.at[b],
                             add=True)
        for b in range(NBUF):
            pltpu.make_async_copy(rows.at[b], acc.at[idx_d.at[j + b]],
                                  sems.at[b]).wait()
            pltpu.async_copy(gh.at[idx_s.at[j + b + NBUF]], rows.at[b],
                             semg.at[b])

    for b in range(NBUF):
        ch = NCH_DEG - NBUF + b
        pltpu.make_async_copy(gh.at[idx_s.at[ch]], rows.at[b],
                              semg.at[b]).wait()
        pltpu.async_copy(rows.at[b], acc.at[idx_d.at[ch]], sems.at[b],
                         add=True)
    for b in range(NBUF):
        pltpu.make_async_copy(rows.at[b], acc.at[idx_d.at[NCH_DEG - NBUF + b]],
                              sems.at[b]).wait()

    plsc.subcore_barrier()
    pltpu.sync_copy(acc.at[pl.ds(s * ROWS_PT, ROWS_PT)],
                    out.at[c, pl.ds(s * ROWS_PT, ROWS_PT)])


@functools.partial(
    pl.kernel,
    out_type=jax.ShapeDtypeStruct((E_PAD,), _f32),
    mesh=_MESH,
    scratch_types=[
        pltpu.VMEM((NCH_E, CHUNK_E), jnp.int32),
        pltpu.VMEM((NCH_E, CHUNK_E), jnp.int32),
        pltpu.VMEM((NBUF, CHUNK_E, HIDDEN), _f32),
        pltpu.VMEM((NBUF, CHUNK_E, HIDDEN), _f32),
        pltpu.VMEM((NBUF, CHUNK_E), _f32),
        pltpu.VMEM((HIDDEN,), _f32),
        pltpu.VMEM((HIDDEN,), _f32),
        pltpu.VMEM((LANES,), _f32),
        pltpu.SemaphoreType.DMA((NBUF,)),
        pltpu.SemaphoreType.DMA((NBUF,)),
        pltpu.SemaphoreType.DMA((NBUF,)),
    ],
)
def _sc_edge(p, ei, b1, w2, b2, out,
             idx_s, idx_d, buf_s, buf_d, res, b1_v, w2_v, b2_v,
             sem_s, sem_d, sem_o):
    c = lax.axis_index("c")
    s = lax.axis_index("s")
    w = c * NS + s
    base = w * EPT_E
    pltpu.sync_copy(ei.at[0, w], idx_s)
    pltpu.sync_copy(ei.at[1, w], idx_d)
    pltpu.sync_copy(b1, b1_v)
    pltpu.sync_copy(w2, w2_v)
    pltpu.sync_copy(b2, b2_v)
    lane = lax.iota(jnp.int32, LANES)
    perms = [jnp.bitwise_xor(lane, sh) for sh in (8, 4, 2, 1)]
    b1s = [b1_v[pl.ds(q * LANES, LANES)] for q in range(HIDDEN // LANES)]
    w2s = [w2_v[pl.ds(q * LANES, LANES)] for q in range(HIDDEN // LANES)]
    b2vec = b2_v[...]

    _dnums = lax.GatherDimensionNumbers(
        offset_dims=(), collapsed_slice_dims=(0,), start_index_map=(0,))

    def hsum(v):
        for perm in perms:
            shuf = lax.gather(v, perm[:, None], _dnums, slice_sizes=(1,),
                              mode=lax.GatherScatterMode.PROMISE_IN_BOUNDS)
            v = v + shuf
        return v

    def issue(ch, b):
        pltpu.async_copy(p.at[idx_s.at[ch]], buf_s.at[b], sem_s.at[b])
        pltpu.async_copy(p.at[idx_d.at[ch]], buf_d.at[b], sem_d.at[b])

    def compute(ch, b, wait_out):
        pltpu.make_async_copy(p.at[idx_s.at[ch]], buf_s.at[b],
                              sem_s.at[b]).wait()
        pltpu.make_async_copy(p.at[idx_d.at[ch]], buf_d.at[b],
                              sem_d.at[b]).wait()
        if wait_out is True:
            pltpu.make_async_copy(res.at[b], out.at[pl.ds(base, CHUNK_E)],
                                  sem_o.at[b]).wait()
        elif wait_out is not None:
            @pl.when(wait_out)
            def _():
                pltpu.make_async_copy(
                    res.at[b], out.at[pl.ds(base, CHUNK_E)], sem_o.at[b]).wait()

        @pl.loop(0, CHUNK_E // LANES)
        def _(gi):
            vout = jnp.zeros((LANES,), _f32)
            for l in range(LANES):
                e = gi * LANES + l
                acc = jnp.zeros((LANES,), _f32)
                for q in range(HIDDEN // LANES):
                    sq = buf_s[b, e, pl.ds(q * LANES, LANES)]
                    dq = buf_d[b, e, pl.ds(q * LANES, LANES)]
                    t = jnp.maximum(sq + dq + b1s[q], 0.0)
                    acc = acc + t * w2s[q]
                vout = jnp.where(lane == l, hsum(acc), vout)
            res[b, pl.ds(gi * LANES, LANES)] = vout + b2vec

        pltpu.async_copy(res.at[b], out.at[pl.ds(base + ch * CHUNK_E, CHUNK_E)],
                         sem_o.at[b])

    for b in range(NBUF):
        issue(b, b)

    @pl.loop(0, NCH_E - NBUF, step=NBUF)
    def _(j):
        for b in range(NBUF):
            compute(j + b, b, wait_out=j + b >= NBUF)
            issue(j + b + NBUF, b)

    for b in range(NBUF):
        compute(NCH_E - NBUF + b, b, wait_out=True)
    for b in range(NBUF):
        pltpu.make_async_copy(res.at[b], out.at[pl.ds(base, CHUNK_E)],
                              sem_o.at[b]).wait()


def _tc_norms_body(deg_ref, nin_ref, nout_ref):
    deg = deg_ref[...]
    nout_ref[...] = lax.rsqrt(jnp.clip(deg[0][:N_NODES, 0:1], 1.0, None))
    nin_ref[...] = lax.rsqrt(jnp.clip(deg[1][:N_NODES, 0:1], 1.0, None))


def _tc_norms(degs):
    return pl.pallas_call(
        _tc_norms_body,
        out_shape=(
            jax.ShapeDtypeStruct((N_NODES, 1), _f32),
            jax.ShapeDtypeStruct((N_NODES, 1), _f32),
        ),
    )(degs)


def _tc_embed_body(nf_ref, wn_ref, bn_ref, nout_ref, g1_ref):
    h0 = jnp.dot(nf_ref[...], wn_ref[...], preferred_element_type=_f32, precision=lax.Precision.HIGHEST)
    h0 = h0 + bn_ref[...]
    g1 = h0 * nout_ref[...]
    g1_ref[0] = g1[:, :HALF]
    g1_ref[1] = g1[:, HALF:]


def _tc_embed(nf, Wn, bn, nout):
    return pl.pallas_call(
        _tc_embed_body,
        out_shape=jax.ShapeDtypeStruct((NC, N_NODES, HALF), _f32),
    )(nf, Wn, bn, nout)


def _tc_layer_body(parts_ref, nin_ref, nout_ref, w_ref, b_ref, out_ref):
    parts = parts_ref[...]
    agg = jnp.concatenate([parts[0, :N_NODES], parts[1, :N_NODES]], axis=1)
    agg = agg * nin_ref[...]
    h = jnp.dot(agg, w_ref[...], preferred_element_type=_f32, precision=lax.Precision.HIGHEST) + b_ref[...]
    g = jnp.maximum(h, 0.0) * nout_ref[...]
    out_ref[0] = g[:, :HALF]
    out_ref[1] = g[:, HALF:]


def _tc_layer(parts, nin, nout, W, b):
    return pl.pallas_call(
        _tc_layer_body,
        out_shape=jax.ShapeDtypeStruct((NC, N_NODES, HALF), _f32),
    )(parts, nin, nout, W, b)


def _tc_final_body(parts_ref, nin_ref, w_ref, b_ref, wo1_ref, out_ref):
    parts = parts_ref[...]
    agg = jnp.concatenate([parts[0, :N_NODES], parts[1, :N_NODES]], axis=1)
    agg = agg * nin_ref[...]
    h = jnp.dot(agg, w_ref[...], preferred_element_type=_f32, precision=lax.Precision.HIGHEST) + b_ref[...]
    h = jnp.maximum(h, 0.0)
    out_ref[...] = jnp.dot(h, wo1_ref[...], preferred_element_type=_f32, precision=lax.Precision.HIGHEST)


def _tc_final(parts, nin, W, b, Wo1):
    return pl.pallas_call(
        _tc_final_body,
        out_shape=jax.ShapeDtypeStruct((N_NODES, HIDDEN), _f32),
    )(parts, nin, W, b, Wo1)


def kernel(edge_feats, node_feats, edge_index, W_e, b_e, W_n, b_n,
           W_g1, b_g1, W_g2, b_g2, W_o1, b_o1, W_o2, b_o2):
    del edge_feats, W_e, b_e
    ei_deg = edge_index.reshape(2, NS, NCH_DEG, CHUNK)
    ei_pad = jnp.concatenate(
        [edge_index, jnp.zeros((2, E_PAD - N_EDGES), jnp.int32)], axis=1)
    ei_lay = ei_pad.reshape(2, NW, NCH_E, CHUNK_E)
    z16 = jnp.zeros((ROWS_PT, DEG_W), _f32)
    zh = jnp.zeros((ROWS_PT, HALF), _f32)
    ones16 = jnp.ones((CHUNK, DEG_W), _f32)

    degs = _sc_degrees(ei_deg, z16, ones16)
    nin, nout = _tc_norms(degs)
    g1 = _tc_embed(node_feats, W_n, b_n.reshape(1, -1), nout)
    parts1 = _sc_aggregate(g1, ei_deg, zh)
    g2 = _tc_layer(parts1, nin, nout, W_g1, b_g1.reshape(1, -1))
    parts2 = _sc_aggregate(g2, ei_deg, zh)
    p = _tc_final(parts2, nin, W_g2, b_g2.reshape(1, -1), W_o1)

    b2v = jnp.full((LANES,), b_o2[0], _f32)
    preds = _sc_edge(p, ei_lay, b_o1, W_o2.reshape(-1), b2v)
    return preds[:N_EDGES, None]


# speedup vs baseline: 1.1307x; 1.1307x over previous
import functools

import jax
import jax.numpy as jnp
from jax import lax
from jax.experimental import pallas as pl
from jax.experimental.pallas import tpu as pltpu
from jax.experimental.pallas import tpu_sc as plsc

N_NODES = 10000
N_EDGES = 320000
HIDDEN = 128

NC = 2
NS = 16
NW = NC * NS
LANES = 16

CHUNK = 80
EPT = N_EDGES // NW
NCH = EPT // CHUNK
EPT_DEG = N_EDGES // NS
NCH_DEG = EPT_DEG // CHUNK
E_PAD = 327680
EPT_E = E_PAD // NW
CHUNK_E = 64
NCH_E = EPT_E // CHUNK_E
N_PAD = 10240
ROWS_PT = N_PAD // NS
DEG_W = 16

_MESH = plsc.VectorSubcoreMesh(core_axis_name="c", subcore_axis_name="s")
_f32 = jnp.float32


@functools.partial(
    pl.kernel,
    out_type=jax.ShapeDtypeStruct((NC, N_PAD, DEG_W), _f32),
    mesh=_MESH,
    scratch_types=[
        pltpu.VMEM((NCH_DEG, CHUNK), jnp.int32),
        pltpu.VMEM((CHUNK, DEG_W), _f32),
        pltpu.VMEM_SHARED((N_PAD, DEG_W), _f32),
    ],
    compiler_params=pltpu.CompilerParams(use_tc_tiling_on_sc=False),
)
def _sc_degrees(ei, z16, ones, out, idx_v, ones_v, acc):
    c = lax.axis_index("c")
    s = lax.axis_index("s")
    pltpu.sync_copy(ei.at[c, s], idx_v)
    pltpu.sync_copy(ones, ones_v)
    pltpu.sync_copy(z16, acc.at[pl.ds(s * ROWS_PT, ROWS_PT)])
    plsc.subcore_barrier()

    @pl.loop(0, NCH_DEG)
    def _(j):
        pltpu.sync_copy(ones_v, acc.at[idx_v.at[j]], add=True)

    plsc.subcore_barrier()
    pltpu.sync_copy(acc.at[pl.ds(s * ROWS_PT, ROWS_PT)],
                    out.at[c, pl.ds(s * ROWS_PT, ROWS_PT)])


NBUF = 5
HALF = HIDDEN // NC


@functools.partial(
    pl.kernel,
    out_type=jax.ShapeDtypeStruct((NC, N_PAD, HALF), _f32),
    mesh=_MESH,
    scratch_types=[
        pltpu.VMEM((NCH_DEG, CHUNK), jnp.int32),
        pltpu.VMEM((NCH_DEG, CHUNK), jnp.int32),
        pltpu.VMEM((NBUF, CHUNK, HALF), _f32),
        pltpu.VMEM_SHARED((N_PAD, HALF), _f32),
        pltpu.SemaphoreType.DMA((NBUF,)),
        pltpu.SemaphoreType.DMA((NBUF,)),
    ],
    compiler_params=pltpu.CompilerParams(use_tc_tiling_on_sc=False),
)
def _sc_aggregate(g, ei, z, out, idx_s, idx_d, rows, acc, semg, sems):
    c = lax.axis_index("c")
    s = lax.axis_index("s")
    gh = g.at[c]
    pltpu.sync_copy(ei.at[0, s], idx_s)
    pltpu.sync_copy(ei.at[1, s], idx_d)
    pltpu.sync_copy(z, acc.at[pl.ds(s * ROWS_PT, ROWS_PT)])
    plsc.subcore_barrier()

    for b in range(NBUF):
        pltpu.async_copy(gh.at[idx_s.at[b]], rows.at[b], semg.at[b])

    @pl.loop(0, NCH_DEG - NBUF, step=NBUF)
    def _(j):
        for b in range(NBUF):
            ch = j + b
            pltpu.make_async_copy(gh.at[idx_s.at[ch]], rows.at[b],
                                  semg.at[b]).wait()
            pltpu.async_copy(rows.at[b], acc.at[idx_d.at[ch]], sems.at[b],
                             add=True)
        for b in range(NBUF):
            pltpu.make_async_copy(rows.at[b], acc.at[idx_d.at[j + b]],
                                  sems.at[b]).wait()
            pltpu.async_copy(gh.at[idx_s.at[j + b + NBUF]], rows.at[b],
                             semg.at[b])

    for b in range(NBUF):
        ch = NCH_DEG - NBUF + b
        pltpu.make_async_copy(gh.at[idx_s.at[ch]], rows.at[b],
                              semg.at[b]).wait()
        pltpu.async_copy(rows.at[b], acc.at[idx_d.at[ch]], sems.at[b],
                         add=True)
    for b in range(NBUF):
        pltpu.make_async_copy(rows.at[b], acc.at[idx_d.at[NCH_DEG - NBUF + b]],
                              sems.at[b]).wait()

    plsc.subcore_barrier()
    pltpu.sync_copy(acc.at[pl.ds(s * ROWS_PT, ROWS_PT)],
                    out.at[c, pl.ds(s * ROWS_PT, ROWS_PT)])


@functools.partial(
    pl.kernel,
    out_type=jax.ShapeDtypeStruct((E_PAD,), _f32),
    mesh=_MESH,
    scratch_types=[
        pltpu.VMEM((NCH_E, CHUNK_E), jnp.int32),
        pltpu.VMEM((NCH_E, CHUNK_E), jnp.int32),
        pltpu.VMEM((NBUF, CHUNK_E, HIDDEN), _f32),
        pltpu.VMEM((NBUF, CHUNK_E, HIDDEN), _f32),
        pltpu.VMEM((NBUF, CHUNK_E), _f32),
        pltpu.VMEM((HIDDEN,), _f32),
        pltpu.VMEM((HIDDEN,), _f32),
        pltpu.VMEM((LANES,), _f32),
        pltpu.SemaphoreType.DMA((NBUF,)),
        pltpu.SemaphoreType.DMA((NBUF,)),
        pltpu.SemaphoreType.DMA((NBUF,)),
    ],
    compiler_params=pltpu.CompilerParams(use_tc_tiling_on_sc=False),
)
def _sc_edge(p, ei, b1, w2, b2, out,
             idx_s, idx_d, buf_s, buf_d, res, b1_v, w2_v, b2_v,
             sem_s, sem_d, sem_o):
    c = lax.axis_index("c")
    s = lax.axis_index("s")
    w = c * NS + s
    base = w * EPT_E
    pltpu.sync_copy(ei.at[0, w], idx_s)
    pltpu.sync_copy(ei.at[1, w], idx_d)
    pltpu.sync_copy(b1, b1_v)
    pltpu.sync_copy(w2, w2_v)
    pltpu.sync_copy(b2, b2_v)
    lane = lax.iota(jnp.int32, LANES)
    perms = [jnp.bitwise_xor(lane, sh) for sh in (8, 4, 2, 1)]
    b1s = [b1_v[pl.ds(q * LANES, LANES)] for q in range(HIDDEN // LANES)]
    w2s = [w2_v[pl.ds(q * LANES, LANES)] for q in range(HIDDEN // LANES)]
    b2vec = b2_v[...]

    _dnums = lax.GatherDimensionNumbers(
        offset_dims=(), collapsed_slice_dims=(0,), start_index_map=(0,))

    def hsum(v):
        for perm in perms:
            shuf = lax.gather(v, perm[:, None], _dnums, slice_sizes=(1,),
                              mode=lax.GatherScatterMode.PROMISE_IN_BOUNDS)
            v = v + shuf
        return v

    def issue(ch, b):
        pltpu.async_copy(p.at[idx_s.at[ch]], buf_s.at[b], sem_s.at[b])
        pltpu.async_copy(p.at[idx_d.at[ch]], buf_d.at[b], sem_d.at[b])

    def compute(ch, b, wait_out):
        pltpu.make_async_copy(p.at[idx_s.at[ch]], buf_s.at[b],
                              sem_s.at[b]).wait()
        pltpu.make_async_copy(p.at[idx_d.at[ch]], buf_d.at[b],
                              sem_d.at[b]).wait()
        if wait_out:
            pltpu.make_async_copy(res.at[b], out.at[pl.ds(base, CHUNK_E)],
                                  sem_o.at[b]).wait()

        @pl.loop(0, CHUNK_E // LANES)
        def _(gi):
            vout = jnp.zeros((LANES,), _f32)
            for l in range(LANES):
                e = gi * LANES + l
                acc = jnp.zeros((LANES,), _f32)
                for q in range(HIDDEN // LANES):
                    sq = buf_s[b, e, pl.ds(q * LANES, LANES)]
                    dq = buf_d[b, e, pl.ds(q * LANES, LANES)]
                    t = jnp.maximum(sq + dq + b1s[q], 0.0)
                    acc = acc + t * w2s[q]
                vout = jnp.where(lane == l, hsum(acc), vout)
            res[b, pl.ds(gi * LANES, LANES)] = vout + b2vec

        pltpu.async_copy(res.at[b], out.at[pl.ds(base + ch * CHUNK_E, CHUNK_E)],
                         sem_o.at[b])

    for b in range(NBUF):
        issue(b, b)
    for b in range(NBUF):
        compute(b, b, wait_out=False)
        issue(b + NBUF, b)

    @pl.loop(NBUF, NCH_E - NBUF, step=NBUF)
    def _(j):
        for b in range(NBUF):
            compute(j + b, b, wait_out=True)
            issue(j + b + NBUF, b)

    for b in range(NBUF):
        compute(NCH_E - NBUF + b, b, wait_out=True)
    for b in range(NBUF):
        pltpu.make_async_copy(res.at[b], out.at[pl.ds(base, CHUNK_E)],
                              sem_o.at[b]).wait()


def _tc_norms_body(deg_ref, nin_ref, nout_ref):
    deg = deg_ref[...]
    nout_ref[...] = lax.rsqrt(jnp.clip(deg[0][:N_NODES, 0:1], 1.0, None))
    nin_ref[...] = lax.rsqrt(jnp.clip(deg[1][:N_NODES, 0:1], 1.0, None))


def _tc_norms(degs):
    return pl.pallas_call(
        _tc_norms_body,
        out_shape=(
            jax.ShapeDtypeStruct((N_NODES, 1), _f32),
            jax.ShapeDtypeStruct((N_NODES, 1), _f32),
        ),
    )(degs)


def _tc_embed_body(nf_ref, wn_ref, bn_ref, nout_ref, g1_ref):
    h0 = jnp.dot(nf_ref[...], wn_ref[...], preferred_element_type=_f32, precision=lax.Precision.HIGHEST)
    h0 = h0 + bn_ref[...]
    g1 = h0 * nout_ref[...]
    g1_ref[0] = g1[:, :HALF]
    g1_ref[1] = g1[:, HALF:]


def _tc_embed(nf, Wn, bn, nout):
    return pl.pallas_call(
        _tc_embed_body,
        out_shape=jax.ShapeDtypeStruct((NC, N_NODES, HALF), _f32),
    )(nf, Wn, bn, nout)


def _tc_layer_body(parts_ref, nin_ref, nout_ref, w_ref, b_ref, out_ref):
    parts = parts_ref[...]
    agg = jnp.concatenate([parts[0, :N_NODES], parts[1, :N_NODES]], axis=1)
    agg = agg * nin_ref[...]
    h = jnp.dot(agg, w_ref[...], preferred_element_type=_f32, precision=lax.Precision.HIGHEST) + b_ref[...]
    g = jnp.maximum(h, 0.0) * nout_ref[...]
    out_ref[0] = g[:, :HALF]
    out_ref[1] = g[:, HALF:]


def _tc_layer(parts, nin, nout, W, b):
    return pl.pallas_call(
        _tc_layer_body,
        out_shape=jax.ShapeDtypeStruct((NC, N_NODES, HALF), _f32),
    )(parts, nin, nout, W, b)


def _tc_final_body(parts_ref, nin_ref, w_ref, b_ref, wo1_ref, out_ref):
    parts = parts_ref[...]
    agg = jnp.concatenate([parts[0, :N_NODES], parts[1, :N_NODES]], axis=1)
    agg = agg * nin_ref[...]
    h = jnp.dot(agg, w_ref[...], preferred_element_type=_f32, precision=lax.Precision.HIGHEST) + b_ref[...]
    h = jnp.maximum(h, 0.0)
    out_ref[...] = jnp.dot(h, wo1_ref[...], preferred_element_type=_f32, precision=lax.Precision.HIGHEST)


def _tc_final(parts, nin, W, b, Wo1):
    return pl.pallas_call(
        _tc_final_body,
        out_shape=jax.ShapeDtypeStruct((N_NODES, HIDDEN), _f32),
    )(parts, nin, W, b, Wo1)


def kernel(edge_feats, node_feats, edge_index, W_e, b_e, W_n, b_n,
           W_g1, b_g1, W_g2, b_g2, W_o1, b_o1, W_o2, b_o2):
    del edge_feats, W_e, b_e
    ei_deg = edge_index.reshape(2, NS, NCH_DEG, CHUNK)
    ei_pad = jnp.concatenate(
        [edge_index, jnp.zeros((2, E_PAD - N_EDGES), jnp.int32)], axis=1)
    ei_lay = ei_pad.reshape(2, NW, NCH_E, CHUNK_E)
    z16 = jnp.zeros((ROWS_PT, DEG_W), _f32)
    zh = jnp.zeros((ROWS_PT, HALF), _f32)
    ones16 = jnp.ones((CHUNK, DEG_W), _f32)

    degs = _sc_degrees(ei_deg, z16, ones16)
    nin, nout = _tc_norms(degs)
    g1 = _tc_embed(node_feats, W_n, b_n.reshape(1, -1), nout)
    parts1 = _sc_aggregate(g1, ei_deg, zh)
    g2 = _tc_layer(parts1, nin, nout, W_g1, b_g1.reshape(1, -1))
    parts2 = _sc_aggregate(g2, ei_deg, zh)
    p = _tc_final(parts2, nin, W_g2, b_g2.reshape(1, -1), W_o1)

    b2v = jnp.full((LANES,), b_o2[0], _f32)
    preds = _sc_edge(p, ei_lay, b_o1, W_o2.reshape(-1), b2v)
    return preds[:N_EDGES, None]


# speedup vs baseline: 2.7610x; 2.4419x over previous
import functools

import jax
import jax.numpy as jnp
from jax import lax
from jax.experimental import pallas as pl
from jax.experimental.pallas import tpu as pltpu
from jax.experimental.pallas import tpu_sc as plsc

N_NODES = 10000
N_EDGES = 320000
HIDDEN = 128

NC = 2
NS = 16
NW = NC * NS
LANES = 16

CHUNK = 80
EPT = N_EDGES // NW
NCH = EPT // CHUNK
EPT_DEG = N_EDGES // NS
NCH_DEG = EPT_DEG // CHUNK
E_PAD = 327680
EPT_E = E_PAD // NW
CHUNK_E = 64
NCH_E = EPT_E // CHUNK_E
N_PAD = 10240
ROWS_PT = N_PAD // NS
DEG_W = 16

_MESH = plsc.VectorSubcoreMesh(core_axis_name="c", subcore_axis_name="s")
_f32 = jnp.float32


@functools.partial(
    pl.kernel,
    out_type=jax.ShapeDtypeStruct((NC, N_PAD, DEG_W), _f32),
    mesh=_MESH,
    scratch_types=[
        pltpu.VMEM((NCH_DEG, CHUNK), jnp.int32),
        pltpu.VMEM((CHUNK, DEG_W), _f32),
        pltpu.VMEM_SHARED((N_PAD, DEG_W), _f32),
    ],
    compiler_params=pltpu.CompilerParams(use_tc_tiling_on_sc=False),
)
def _sc_degrees(ei, z16, ones, out, idx_v, ones_v, acc):
    c = lax.axis_index("c")
    s = lax.axis_index("s")
    pltpu.sync_copy(ei.at[c, s], idx_v)
    pltpu.sync_copy(ones, ones_v)
    pltpu.sync_copy(z16, acc.at[pl.ds(s * ROWS_PT, ROWS_PT)])
    plsc.subcore_barrier()

    @pl.loop(0, NCH_DEG)
    def _(j):
        pltpu.sync_copy(ones_v, acc.at[idx_v.at[j]], add=True)

    plsc.subcore_barrier()
    pltpu.sync_copy(acc.at[pl.ds(s * ROWS_PT, ROWS_PT)],
                    out.at[c, pl.ds(s * ROWS_PT, ROWS_PT)])


NBUF = 5
HALF = HIDDEN // NC


@functools.partial(
    pl.kernel,
    out_type=jax.ShapeDtypeStruct((NC, N_PAD, HALF), _f32),
    mesh=_MESH,
    scratch_types=[
        pltpu.VMEM((NCH_DEG, CHUNK), jnp.int32),
        pltpu.VMEM((NCH_DEG, CHUNK), jnp.int32),
        pltpu.VMEM((NBUF, CHUNK, HALF), _f32),
        pltpu.VMEM_SHARED((N_PAD, HALF), _f32),
        pltpu.SemaphoreType.DMA((NBUF,)),
        pltpu.SemaphoreType.DMA((NBUF,)),
    ],
    compiler_params=pltpu.CompilerParams(use_tc_tiling_on_sc=False),
)
def _sc_aggregate(g, ei, z, out, idx_s, idx_d, rows, acc, semg, sems):
    c = lax.axis_index("c")
    s = lax.axis_index("s")
    gh = g.at[c]
    pltpu.sync_copy(ei.at[0, s], idx_s)
    pltpu.sync_copy(ei.at[1, s], idx_d)
    pltpu.sync_copy(z, acc.at[pl.ds(s * ROWS_PT, ROWS_PT)])
    plsc.subcore_barrier()

    for b in range(NBUF):
        pltpu.async_copy(gh.at[idx_s.at[b]], rows.at[b], semg.at[b])

    @pl.loop(0, NCH_DEG - NBUF, step=NBUF)
    def _(j):
        for b in range(NBUF):
            ch = j + b
            pltpu.make_async_copy(gh.at[idx_s.at[ch]], rows.at[b],
                                  semg.at[b]).wait()
            pltpu.async_copy(rows.at[b], acc.at[idx_d.at[ch]], sems.at[b],
                             add=True)
        for b in range(NBUF):
            pltpu.make_async_copy(rows.at[b], acc.at[idx_d.at[j + b]],
                                  sems.at[b]).wait()
            pltpu.async_copy(gh.at[idx_s.at[j + b + NBUF]], rows.at[b],
                             semg.at[b])

    for b in range(NBUF):
        ch = NCH_DEG - NBUF + b
        pltpu.make_async_copy(gh.at[idx_s.at[ch]], rows.at[b],
                              semg.at[b]).wait()
        pltpu.async_copy(rows.at[b], acc.at[idx_d.at[ch]], sems.at[b],
                         add=True)
    for b in range(NBUF):
        pltpu.make_async_copy(rows.at[b], acc.at[idx_d.at[NCH_DEG - NBUF + b]],
                              sems.at[b]).wait()

    plsc.subcore_barrier()
    pltpu.sync_copy(acc.at[pl.ds(s * ROWS_PT, ROWS_PT)],
                    out.at[c, pl.ds(s * ROWS_PT, ROWS_PT)])


@functools.partial(
    pl.kernel,
    out_type=jax.ShapeDtypeStruct((E_PAD,), _f32),
    mesh=_MESH,
    scratch_types=[
        pltpu.VMEM((NCH_E, CHUNK_E), jnp.int32),
        pltpu.VMEM((NCH_E, CHUNK_E), jnp.int32),
        pltpu.VMEM((NBUF, CHUNK_E, HIDDEN), _f32),
        pltpu.VMEM((NBUF, CHUNK_E, HIDDEN), _f32),
        pltpu.VMEM((NBUF, CHUNK_E), _f32),
        pltpu.VMEM((HIDDEN,), _f32),
        pltpu.VMEM((HIDDEN,), _f32),
        pltpu.VMEM((LANES,), _f32),
        pltpu.SemaphoreType.DMA((NBUF,)),
        pltpu.SemaphoreType.DMA((NBUF,)),
        pltpu.SemaphoreType.DMA((NBUF,)),
    ],
    compiler_params=pltpu.CompilerParams(use_tc_tiling_on_sc=False),
)
def _sc_edge(p, ei, b1, w2, b2, out,
             idx_s, idx_d, buf_s, buf_d, res, b1_v, w2_v, b2_v,
             sem_s, sem_d, sem_o):
    c = lax.axis_index("c")
    s = lax.axis_index("s")
    w = c * NS + s
    base = w * EPT_E
    pltpu.sync_copy(ei.at[0, w], idx_s)
    pltpu.sync_copy(ei.at[1, w], idx_d)
    pltpu.sync_copy(b1, b1_v)
    pltpu.sync_copy(w2, w2_v)
    pltpu.sync_copy(b2, b2_v)
    lane = lax.iota(jnp.int32, LANES)
    perms = [jnp.bitwise_xor(lane, sh) for sh in (8, 4, 2, 1)]
    b1s = [b1_v[pl.ds(q * LANES, LANES)] for q in range(HIDDEN // LANES)]
    w2s = [w2_v[pl.ds(q * LANES, LANES)] for q in range(HIDDEN // LANES)]
    b2vec = b2_v[...]

    _dnums = lax.GatherDimensionNumbers(
        offset_dims=(), collapsed_slice_dims=(0,), start_index_map=(0,))

    def hsum(v):
        for perm in perms:
            shuf = lax.gather(v, perm[:, None], _dnums, slice_sizes=(1,),
                              mode=lax.GatherScatterMode.PROMISE_IN_BOUNDS)
            v = v + shuf
        return v

    def issue(ch, b):
        pltpu.async_copy(p.at[idx_s.at[ch]], buf_s.at[b], sem_s.at[b])
        pltpu.async_copy(p.at[idx_d.at[ch]], buf_d.at[b], sem_d.at[b])

    def compute(ch, b, wait_out):
        pltpu.make_async_copy(p.at[idx_s.at[ch]], buf_s.at[b],
                              sem_s.at[b]).wait()
        pltpu.make_async_copy(p.at[idx_d.at[ch]], buf_d.at[b],
                              sem_d.at[b]).wait()
        if wait_out:
            pltpu.make_async_copy(res.at[b], out.at[pl.ds(base, CHUNK_E)],
                                  sem_o.at[b]).wait()

        @pl.loop(0, CHUNK_E // LANES)
        def _(gi):
            vout = jnp.zeros((LANES,), _f32)
            for l in range(LANES):
                e = gi * LANES + l
                acc = jnp.zeros((LANES,), _f32)
                for q in range(HIDDEN // LANES):
                    sq = buf_s[b, e, pl.ds(q * LANES, LANES)]
                    dq = buf_d[b, e, pl.ds(q * LANES, LANES)]
                    t = jnp.maximum(sq + dq + b1s[q], 0.0)
                    acc = acc + t * w2s[q]
                vout = jnp.where(lane == l, hsum(acc), vout)
            res[b, pl.ds(gi * LANES, LANES)] = vout + b2vec

        pltpu.async_copy(res.at[b], out.at[pl.ds(base + ch * CHUNK_E, CHUNK_E)],
                         sem_o.at[b])

    for b in range(NBUF):
        issue(b, b)
    for b in range(NBUF):
        compute(b, b, wait_out=False)
        issue(b + NBUF, b)

    @pl.loop(NBUF, NCH_E - NBUF, step=NBUF)
    def _(j):
        for b in range(NBUF):
            compute(j + b, b, wait_out=True)
            issue(j + b + NBUF, b)

    for b in range(NBUF):
        compute(NCH_E - NBUF + b, b, wait_out=True)
    for b in range(NBUF):
        pltpu.make_async_copy(res.at[b], out.at[pl.ds(base, CHUNK_E)],
                              sem_o.at[b]).wait()


def _tc_norms_body(deg_ref, nin_ref, nout_ref):
    deg = deg_ref[...]
    nout_ref[...] = lax.rsqrt(jnp.clip(deg[0][:N_NODES, 0:1], 1.0, None))
    nin_ref[...] = lax.rsqrt(jnp.clip(deg[1][:N_NODES, 0:1], 1.0, None))


def _tc_norms(degs):
    return pl.pallas_call(
        _tc_norms_body,
        out_shape=(
            jax.ShapeDtypeStruct((N_NODES, 1), _f32),
            jax.ShapeDtypeStruct((N_NODES, 1), _f32),
        ),
    )(degs)


def _bf16_dot(x, w):
    return jnp.dot(x.astype(jnp.bfloat16), w.astype(jnp.bfloat16),
                   preferred_element_type=_f32)


def _tc_embed_body(nf_ref, wn_ref, bn_ref, nout_ref, g1_ref):
    h0 = _bf16_dot(nf_ref[...], wn_ref[...])
    h0 = h0 + bn_ref[...]
    g1 = h0 * nout_ref[...]
    g1_ref[0] = g1[:, :HALF]
    g1_ref[1] = g1[:, HALF:]


def _tc_embed(nf, Wn, bn, nout):
    return pl.pallas_call(
        _tc_embed_body,
        out_shape=jax.ShapeDtypeStruct((NC, N_NODES, HALF), _f32),
    )(nf, Wn, bn, nout)


def _tc_layer_body(parts_ref, nin_ref, nout_ref, w_ref, b_ref, out_ref):
    parts = parts_ref[...]
    agg = jnp.concatenate([parts[0, :N_NODES], parts[1, :N_NODES]], axis=1)
    agg = agg * nin_ref[...]
    h = _bf16_dot(agg, w_ref[...]) + b_ref[...]
    g = jnp.maximum(h, 0.0) * nout_ref[...]
    out_ref[0] = g[:, :HALF]
    out_ref[1] = g[:, HALF:]


def _tc_layer(parts, nin, nout, W, b):
    return pl.pallas_call(
        _tc_layer_body,
        out_shape=jax.ShapeDtypeStruct((NC, N_NODES, HALF), _f32),
    )(parts, nin, nout, W, b)


def _tc_final_body(parts_ref, nin_ref, w_ref, b_ref, wo1_ref, out_ref):
    parts = parts_ref[...]
    agg = jnp.concatenate([parts[0, :N_NODES], parts[1, :N_NODES]], axis=1)
    agg = agg * nin_ref[...]
    h = _bf16_dot(agg, w_ref[...]) + b_ref[...]
    h = jnp.maximum(h, 0.0)
    out_ref[...] = _bf16_dot(h, wo1_ref[...])


def _tc_final(parts, nin, W, b, Wo1):
    return pl.pallas_call(
        _tc_final_body,
        out_shape=jax.ShapeDtypeStruct((N_NODES, HIDDEN), _f32),
    )(parts, nin, W, b, Wo1)


def kernel(edge_feats, node_feats, edge_index, W_e, b_e, W_n, b_n,
           W_g1, b_g1, W_g2, b_g2, W_o1, b_o1, W_o2, b_o2):
    del edge_feats, W_e, b_e
    ei_deg = edge_index.reshape(2, NS, NCH_DEG, CHUNK)
    pad_ids = jnp.arange(E_PAD - N_EDGES, dtype=jnp.int32) % N_NODES
    ei_pad = jnp.concatenate(
        [edge_index, jnp.stack([pad_ids, pad_ids])], axis=1)
    ei_lay = ei_pad.reshape(2, NW, NCH_E, CHUNK_E)
    z16 = jnp.zeros((ROWS_PT, DEG_W), _f32)
    zh = jnp.zeros((ROWS_PT, HALF), _f32)
    ones16 = jnp.ones((CHUNK, DEG_W), _f32)

    degs = _sc_degrees(ei_deg, z16, ones16)
    nin, nout = _tc_norms(degs)
    g1 = _tc_embed(node_feats, W_n, b_n.reshape(1, -1), nout)
    parts1 = _sc_aggregate(g1, ei_deg, zh)
    g2 = _tc_layer(parts1, nin, nout, W_g1, b_g1.reshape(1, -1))
    parts2 = _sc_aggregate(g2, ei_deg, zh)
    p = _tc_final(parts2, nin, W_g2, b_g2.reshape(1, -1), W_o1)

    b2v = jnp.full((LANES,), b_o2[0], _f32)
    preds = _sc_edge(p, ei_lay, b_o1, W_o2.reshape(-1), b2v)
    return preds[:N_EDGES, None]
